# baseline (speedup 1.0000x reference)
import functools

import jax
import jax.numpy as jnp
from jax import lax
from jax.experimental import pallas as pl
from jax.experimental.pallas import tpu as pltpu
from jax.experimental.pallas import tpu_sc as plsc

N = 100000
E = 6400000
D = 7
DP = 8
NC = 2
NS = 16
W = NC * NS
CHUNK = 128
GROUP = 16
EG = CHUNK * GROUP
NGROUPS = E // EG
NP_ = 100096
ROWS_PER_TILE = NP_ // NS
NROWBLK = E // CHUNK


def _spmm_body(q_hbm, kidx_hbm, kv_hbm, zeros_hbm, out_hbm,
               acc, colv0, colv1, rowv0, rowv1, kvv0, kvv1,
               gath0, gath1, scal0, scal1,
               isem0, isem1, gsem0, gsem1, ssem0, ssem1):
  c = lax.axis_index("c")
  s = lax.axis_index("s")
  wid = s * NC + c
  colv = (colv0, colv1)
  rowv = (rowv0, rowv1)
  kvv = (kvv0, kvv1)
  gath = (gath0, gath1)
  scal = (scal0, scal1)
  isem = (isem0, isem1)
  gsem = (gsem0, gsem1)
  ssem = (ssem0, ssem1)

  rbase = s * ROWS_PER_TILE
  pltpu.sync_copy(zeros_hbm, acc.at[pl.ds(rbase, ROWS_PER_TILE)])
  plsc.subcore_barrier()

  patt = lax.shift_right_logical(lax.iota(jnp.int32, 16), 3)
  colc = lax.iota(jnp.int32, 16) & 7
  nit = (NGROUPS // W) + jnp.where(wid < (NGROUPS % W), 1, 0)

  def fire_idx(b, it):
    eb = (wid + it * W) * EG
    pltpu.async_copy(kidx_hbm.at[1, pl.ds(eb, EG)], colv[b], isem[b])
    pltpu.async_copy(kidx_hbm.at[0, pl.ds(eb, EG)], rowv[b], isem[b])
    pltpu.async_copy(kv_hbm.at[pl.ds(eb, EG)], kvv[b], isem[b])

  def wait_idx(b):
    pltpu.make_async_copy(kidx_hbm.at[1, pl.ds(0, EG)], colv[b], isem[b]).wait()
    pltpu.make_async_copy(kidx_hbm.at[0, pl.ds(0, EG)], rowv[b], isem[b]).wait()
    pltpu.make_async_copy(kv_hbm.at[pl.ds(0, EG)], kvv[b], isem[b]).wait()

  def fire_gather(b):
    pltpu.async_copy(q_hbm.at[colv[b]], gath[b], gsem[b])

  def wait_gather(b):
    pltpu.make_async_copy(q_hbm.at[colv[b]], gath[b], gsem[b]).wait()

  def fire_scatter(b):
    pltpu.async_copy(scal[b], acc.at[rowv[b]], ssem[b], add=True)

  def wait_scatter(b):
    pltpu.make_async_copy(scal[b], acc.at[rowv[b]], ssem[b]).wait()

  @pl.when(nit > 0)
  def _():
    fire_idx(0, 0)
    wait_idx(0)
    fire_gather(0)

  def body2(it2, _):
    for b in (0, 1):
      it = 2 * it2 + b
      ob = 1 - b

      @pl.when(it < nit)
      def _():
        @pl.when(it >= 1)
        def _():
          wait_scatter(ob)

        @pl.when(it + 1 < nit)
        def _():
          fire_idx(ob, it + 1)

        wait_gather(b)

        @plsc.parallel_loop(0, EG // 2, unroll=16)
        def mul_body(i):
          idx2 = patt + 2 * i
          gv = plsc.load_gather(gath[b], [idx2, colc])
          kvb = plsc.load_gather(kvv[b], [idx2])
          plsc.store_scatter(scal[b], [idx2, colc], gv * kvb)

        fire_scatter(b)

        @pl.when(it + 1 < nit)
        def _():
          wait_idx(ob)
          fire_gather(ob)
    return 0

  lax.fori_loop(0, (nit + 1) // 2, body2, 0)

  @pl.when(nit % 2 == 1)
  def _():
    wait_scatter(0)

  @pl.when((nit % 2 == 0) & (nit > 0))
  def _():
    wait_scatter(1)

  plsc.subcore_barrier()
  pltpu.sync_copy(acc.at[pl.ds(rbase, ROWS_PER_TILE)],
                  out_hbm.at[c, pl.ds(rbase, ROWS_PER_TILE)])


_sc_spmm = functools.partial(
    pl.kernel,
    out_type=jax.ShapeDtypeStruct((NC, NP_, DP), jnp.float32),
    mesh=plsc.VectorSubcoreMesh(core_axis_name="c", subcore_axis_name="s"),
    compiler_params=pltpu.CompilerParams(
        needs_layout_passes=False, use_tc_tiling_on_sc=False),
    scratch_types=[
        pltpu.VMEM_SHARED((NP_, DP), jnp.float32),
        pltpu.VMEM((EG,), jnp.int32),
        pltpu.VMEM((EG,), jnp.int32),
        pltpu.VMEM((EG,), jnp.int32),
        pltpu.VMEM((EG,), jnp.int32),
        pltpu.VMEM((EG,), jnp.float32),
        pltpu.VMEM((EG,), jnp.float32),
        pltpu.VMEM((EG, DP), jnp.float32),
        pltpu.VMEM((EG, DP), jnp.float32),
        pltpu.VMEM((EG, DP), jnp.float32),
        pltpu.VMEM((EG, DP), jnp.float32),
        pltpu.SemaphoreType.DMA,
        pltpu.SemaphoreType.DMA,
        pltpu.SemaphoreType.DMA,
        pltpu.SemaphoreType.DMA,
        pltpu.SemaphoreType.DMA,
        pltpu.SemaphoreType.DMA,
    ],
)(_spmm_body)


BLK = 4000
GRID = N // BLK


def _softmax_body(x_ref, o_ref):
  x = x_ref[...]
  m = jnp.max(x, axis=1, keepdims=True)
  e = jnp.exp(x - m)
  sm = e / jnp.sum(e, axis=1, keepdims=True)
  o_ref[...] = jnp.pad(sm, ((0, 0), (0, DP - D)))


def _compat(w):
  eye = (lax.broadcasted_iota(jnp.int32, (DP, DP), 0)
         == lax.broadcasted_iota(jnp.int32, (DP, DP), 1))
  return w * jnp.where(eye, 1.0, -1.0)


def _dense_body(x_ref, a_ref, b_ref, w_ref, o_ref):
  mp_ = a_ref[0] + b_ref[0]
  o_ref[...] = x_ref[...] - jnp.dot(mp_, _compat(w_ref[...]),
                                    preferred_element_type=jnp.float32)


def _final_body(q0_ref, x_ref, a_ref, b_ref, w_ref, o_ref):
  mp_ = a_ref[0] + b_ref[0]
  q1 = x_ref[...] - jnp.dot(mp_, _compat(w_ref[...]),
                            preferred_element_type=jnp.float32)
  t = q0_ref[...] + q1
  tm = t[:, :D]
  m = jnp.max(tm, axis=1, keepdims=True)
  e = jnp.exp(tm - m)
  o_ref[...] = e / jnp.sum(e, axis=1, keepdims=True)


_row_spec = pl.BlockSpec((BLK, DP), lambda i: (i, 0))
_rowD_spec = pl.BlockSpec((BLK, D), lambda i: (i, 0))
_a0_spec = pl.BlockSpec((1, BLK, DP), lambda i: (0, i, 0))
_a1_spec = pl.BlockSpec((1, BLK, DP), lambda i: (1, i, 0))
_w_spec = pl.BlockSpec((DP, DP), lambda i: (0, 0))
_out_sds = jax.ShapeDtypeStruct((N, DP), jnp.float32)

_tc_softmax = pl.pallas_call(
    _softmax_body, grid=(GRID,), in_specs=[_rowD_spec],
    out_specs=_row_spec, out_shape=_out_sds)

_tc_dense = pl.pallas_call(
    _dense_body, grid=(GRID,),
    in_specs=[_row_spec, _a0_spec, _a1_spec, _w_spec],
    out_specs=_row_spec, out_shape=_out_sds)

_tc_final = pl.pallas_call(
    _final_body, grid=(GRID,),
    in_specs=[_row_spec, _row_spec, _a0_spec, _a1_spec, _w_spec],
    out_specs=pl.BlockSpec((BLK, D), lambda i: (i, 0)),
    out_shape=jax.ShapeDtypeStruct((N, D), jnp.float32))


def kernel(inputs, kernel_values, W3, kernel_indices):
  x8 = jnp.pad(inputs, ((0, 0), (0, DP - D)))
  w8 = jnp.pad(W3, ((0, 0), (0, DP - D), (0, DP - D)))
  zeros = jnp.zeros((ROWS_PER_TILE, DP), jnp.float32)

  q_soft = _tc_softmax(inputs)
  a = _sc_spmm(q_soft, kernel_indices, kernel_values, zeros)
  q0 = _tc_dense(x8, a, a, w8[0])
  b = _sc_spmm(q0, kernel_indices, kernel_values, zeros)
  return _tc_final(q0, x8, b, b, w8[1])


# speedup vs baseline: 1.2640x; 1.2640x over previous
import functools

import jax
import jax.numpy as jnp
from jax import lax
from jax.experimental import pallas as pl
from jax.experimental.pallas import tpu as pltpu
from jax.experimental.pallas import tpu_sc as plsc

N = 100000
E = 6400000
D = 7
DP = 8
NC = 2
NS = 16
W = NC * NS
CHUNK = 128
GROUP = 16
EG = CHUNK * GROUP
NGROUPS = E // EG
NP_ = 100096
ROWS_PER_TILE = NP_ // NS
NROWBLK = E // CHUNK


def _spmm_body(q_hbm, kidx_hbm, kv_hbm, zeros_hbm, out_hbm,
               acc, colv0, colv1, rowv0, rowv1, kvv0, kvv1,
               gath0, gath1, scal0, scal1,
               isem0, isem1, gsem0, gsem1, ssem0, ssem1):
  c = lax.axis_index("c")
  s = lax.axis_index("s")
  wid = s * NC + c
  colv = (colv0, colv1)
  rowv = (rowv0, rowv1)
  kvv = (kvv0, kvv1)
  gath = (gath0, gath1)
  scal = (scal0, scal1)
  isem = (isem0, isem1)
  gsem = (gsem0, gsem1)
  ssem = (ssem0, ssem1)

  rbase = s * ROWS_PER_TILE
  pltpu.sync_copy(zeros_hbm, acc.at[pl.ds(rbase, ROWS_PER_TILE)])
  plsc.subcore_barrier()

  patt = lax.shift_right_logical(lax.iota(jnp.int32, 16), 3)
  colc = lax.iota(jnp.int32, 16) & 7
  nit = (NGROUPS // W) + jnp.where(wid < (NGROUPS % W), 1, 0)

  def fire_idx(b, it):
    eb = (wid + it * W) * EG
    pltpu.async_copy(kidx_hbm.at[1, pl.ds(eb, EG)], colv[b], isem[b])
    pltpu.async_copy(kidx_hbm.at[0, pl.ds(eb, EG)], rowv[b], isem[b])
    pltpu.async_copy(kv_hbm.at[pl.ds(eb, EG)], kvv[b], isem[b])

  def wait_idx(b):
    pltpu.make_async_copy(kidx_hbm.at[1, pl.ds(0, EG)], colv[b], isem[b]).wait()
    pltpu.make_async_copy(kidx_hbm.at[0, pl.ds(0, EG)], rowv[b], isem[b]).wait()
    pltpu.make_async_copy(kv_hbm.at[pl.ds(0, EG)], kvv[b], isem[b]).wait()

  def fire_gather(b):
    pltpu.async_copy(q_hbm.at[colv[b]], gath[b], gsem[b])

  def wait_gather(b):
    pltpu.make_async_copy(q_hbm.at[colv[b]], gath[b], gsem[b]).wait()

  def fire_scatter(b):
    pltpu.async_copy(scal[b], acc.at[rowv[b]], ssem[b], add=True)

  def wait_scatter(b):
    pltpu.make_async_copy(scal[b], acc.at[rowv[b]], ssem[b]).wait()

  @pl.when(nit > 0)
  def _():
    fire_idx(0, 0)
    wait_idx(0)
    fire_gather(0)

  def body2(it2, _):
    for b in (0, 1):
      it = 2 * it2 + b
      ob = 1 - b

      @pl.when(it < nit)
      def _():
        @pl.when(it >= 1)
        def _():
          wait_scatter(ob)

        @pl.when(it + 1 < nit)
        def _():
          fire_idx(ob, it + 1)

        wait_gather(b)

        @plsc.parallel_loop(0, EG // 2, unroll=8)
        def mul_body(i):
          idx2 = patt + 2 * i
          gv = plsc.load_gather(gath[b], [idx2, colc])
          kvb = plsc.load_gather(kvv[b], [idx2])
          plsc.store_scatter(scal[b], [idx2, colc], gv * kvb)

        fire_scatter(b)

        @pl.when(it + 1 < nit)
        def _():
          wait_idx(ob)
          fire_gather(ob)
    return 0

  lax.fori_loop(0, (nit + 1) // 2, body2, 0)

  @pl.when(nit % 2 == 1)
  def _():
    wait_scatter(0)

  @pl.when((nit % 2 == 0) & (nit > 0))
  def _():
    wait_scatter(1)

  plsc.subcore_barrier()
  pltpu.sync_copy(acc.at[pl.ds(rbase, ROWS_PER_TILE)],
                  out_hbm.at[c, pl.ds(rbase, ROWS_PER_TILE)])


_sc_spmm = functools.partial(
    pl.kernel,
    out_type=jax.ShapeDtypeStruct((NC, NP_, DP), jnp.float32),
    mesh=plsc.VectorSubcoreMesh(core_axis_name="c", subcore_axis_name="s"),
    compiler_params=pltpu.CompilerParams(
        needs_layout_passes=False, use_tc_tiling_on_sc=False),
    scratch_types=[
        pltpu.VMEM_SHARED((NP_, DP), jnp.float32),
        pltpu.VMEM((EG,), jnp.int32),
        pltpu.VMEM((EG,), jnp.int32),
        pltpu.VMEM((EG,), jnp.int32),
        pltpu.VMEM((EG,), jnp.int32),
        pltpu.VMEM((EG,), jnp.float32),
        pltpu.VMEM((EG,), jnp.float32),
        pltpu.VMEM((EG, DP), jnp.float32),
        pltpu.VMEM((EG, DP), jnp.float32),
        pltpu.VMEM((EG, DP), jnp.float32),
        pltpu.VMEM((EG, DP), jnp.float32),
        pltpu.SemaphoreType.DMA,
        pltpu.SemaphoreType.DMA,
        pltpu.SemaphoreType.DMA,
        pltpu.SemaphoreType.DMA,
        pltpu.SemaphoreType.DMA,
        pltpu.SemaphoreType.DMA,
    ],
)(_spmm_body)


NR = N // 16
BLK16 = 625
GRID16 = NR // BLK16


def _lane_mask7(shape):
  lane = lax.broadcasted_iota(jnp.int32, shape, 1)
  return (lane & 7) < D


def _bd8():
  r = lax.broadcasted_iota(jnp.int32, (128, 128), 0)
  c = lax.broadcasted_iota(jnp.int32, (128, 128), 1)
  return jnp.where((r >> 3) == (c >> 3), 1.0, 0.0)


def _bd_compat(w):
  cm = _compat(w)
  big = jnp.tile(cm, (16, 16))
  r = lax.broadcasted_iota(jnp.int32, (128, 128), 0)
  c = lax.broadcasted_iota(jnp.int32, (128, 128), 1)
  return jnp.where((r >> 3) == (c >> 3), big, 0.0)


def _softmax_body(x_ref, o_ref):
  x = x_ref[...]
  e = jnp.where(_lane_mask7(x.shape), jnp.exp(x), 0.0)
  s = jnp.dot(e, _bd8(), preferred_element_type=jnp.float32)
  o_ref[...] = e / s


def _compat(w):
  eye = (lax.broadcasted_iota(jnp.int32, (DP, DP), 0)
         == lax.broadcasted_iota(jnp.int32, (DP, DP), 1))
  return w * jnp.where(eye, 1.0, -1.0)


def _dense_body(x_ref, a_ref, w_ref, o_ref):
  mp_ = a_ref[0, :NR] + a_ref[1, :NR]
  o_ref[...] = x_ref[...] - jnp.dot(mp_, _bd_compat(w_ref[...]),
                                    preferred_element_type=jnp.float32)


def _final_body(q0_ref, x_ref, a_ref, w_ref, o_ref):
  mp_ = a_ref[0, :NR] + a_ref[1, :NR]
  q1 = x_ref[...] - jnp.dot(mp_, _bd_compat(w_ref[...]),
                            preferred_element_type=jnp.float32)
  t = q0_ref[...] + q1
  e = jnp.where(_lane_mask7(t.shape), jnp.exp(t), 0.0)
  s = jnp.dot(e, _bd8(), preferred_element_type=jnp.float32)
  o_ref[...] = e / s


_out_sds = jax.ShapeDtypeStruct((NR, 128), jnp.float32)

_tc_softmax = pl.pallas_call(_softmax_body, out_shape=_out_sds)

_tc_dense = pl.pallas_call(_dense_body, out_shape=_out_sds)

_tc_final = pl.pallas_call(_final_body, out_shape=_out_sds)


def kernel(inputs, kernel_values, W3, kernel_indices):
  x16 = jnp.pad(inputs, ((0, 0), (0, DP - D))).reshape(NR, 128)
  w8 = jnp.pad(W3, ((0, 0), (0, DP - D), (0, DP - D)))
  zeros = jnp.zeros((ROWS_PER_TILE, DP), jnp.float32)

  q_soft = _tc_softmax(x16)
  a = _sc_spmm(q_soft.reshape(N, DP), kernel_indices, kernel_values, zeros)
  a16 = a.reshape(NC, NP_ // 16, 128)
  q0 = _tc_dense(x16, a16, w8[0])
  b = _sc_spmm(q0.reshape(N, DP), kernel_indices, kernel_values, zeros)
  b16 = b.reshape(NC, NP_ // 16, 128)
  out16 = _tc_final(q0, x16, b16, w8[1])
  return out16.reshape(N, DP)[:, :D]
